# baseline (device time: 119580 ns/iter reference)
import functools

import jax
import jax.numpy as jnp
from jax import lax
from jax.experimental import pallas as pl
from jax.experimental.pallas import tpu as pltpu

M = 2048
N = 2048
K = 8192
BLK = M // 4
NC = 16
CW = N // NC


def _body(dy_hbm, w_hbm, out_ref,
          dyv, wbuf, partial_ref, xrecv_ref,
          dysem, wsems,
          xsend_sems, xrecv_sems, ysend_sems, yrecv_sems,
          z0send_sems, z0recv_sems, fsend_sems, frecv_sems):
    xi = lax.axis_index("x")
    yi = lax.axis_index("y")
    zi = lax.axis_index("z")
    r = 2 * yi + zi
    r2 = 2 * (1 - yi) + zi
    r3 = 2 * yi + (1 - zi)
    x_peer = (1 - xi, yi, zi)
    y_peer = (xi, 1 - yi, zi)
    z_peer = (xi, yi, 1 - zi)

    def col(c):
        return pl.ds(c * CW, CW)

    rows_r = pl.ds(r * BLK, BLK)
    rows_r2 = pl.ds(r2 * BLK, BLK)
    rows_r3 = pl.ds(r3 * BLK, BLK)

    dy_cp = pltpu.make_async_copy(dy_hbm.at[rows_r], dyv, dysem)
    dy_cp.start()
    w_cps = [None] * NC

    def start_w(c):
        w_cps[c] = pltpu.make_async_copy(
            w_hbm.at[col(c)], wbuf.at[c % 2], wsems.at[c % 2])
        w_cps[c].start()

    start_w(0)
    start_w(1)

    barrier = pltpu.get_barrier_semaphore()
    for p in (x_peer, y_peer, z_peer):
        pl.semaphore_signal(barrier, inc=1, device_id=p,
                            device_id_type=pl.DeviceIdType.MESH)
    pl.semaphore_wait(barrier, 3)

    dy_cp.wait()

    rx, ry, rz0 = [], [], []
    rfwd = [None] * NC

    def drain_x(c):
        rx[c].wait_recv()
        out_ref[rows_r, col(c)] = (partial_ref[:, col(c)]
                                   + xrecv_ref[:, col(c)])
        d = pltpu.make_async_remote_copy(
            src_ref=out_ref.at[rows_r, col(c)],
            dst_ref=out_ref.at[rows_r, col(c)],
            send_sem=ysend_sems.at[c], recv_sem=yrecv_sems.at[c],
            device_id=y_peer, device_id_type=pl.DeviceIdType.MESH)
        d.start()
        ry.append(d)
        d = pltpu.make_async_remote_copy(
            src_ref=out_ref.at[rows_r, col(c)],
            dst_ref=out_ref.at[rows_r, col(c)],
            send_sem=z0send_sems.at[c], recv_sem=z0recv_sems.at[c],
            device_id=z_peer, device_id_type=pl.DeviceIdType.MESH)
        d.start()
        rz0.append(d)

    def drain_gather(c):
        if c % 2 == 0:
            ry[c].wait_recv()
            d = pltpu.make_async_remote_copy(
                src_ref=out_ref.at[rows_r2, col(c)],
                dst_ref=out_ref.at[rows_r2, col(c)],
                send_sem=fsend_sems.at[c], recv_sem=frecv_sems.at[c],
                device_id=z_peer, device_id_type=pl.DeviceIdType.MESH)
        else:
            rz0[c].wait_recv()
            d = pltpu.make_async_remote_copy(
                src_ref=out_ref.at[rows_r3, col(c)],
                dst_ref=out_ref.at[rows_r3, col(c)],
                send_sem=fsend_sems.at[c], recv_sem=frecv_sems.at[c],
                device_id=y_peer, device_id_type=pl.DeviceIdType.MESH)
        d.start()
        rfwd[c] = d

    for c in range(NC):
        w_cps[c].wait()
        partial_ref[:, col(c)] = lax.dot_general(
            dyv[...], wbuf[c % 2],
            dimension_numbers=(((1,), (1,)), ((), ())),
            preferred_element_type=jnp.float32)
        if c + 2 < NC:
            start_w(c + 2)
        d = pltpu.make_async_remote_copy(
            src_ref=partial_ref.at[:, col(c)],
            dst_ref=xrecv_ref.at[:, col(c)],
            send_sem=xsend_sems.at[c], recv_sem=xrecv_sems.at[c],
            device_id=x_peer, device_id_type=pl.DeviceIdType.MESH)
        d.start()
        rx.append(d)
        if c >= 1:
            drain_x(c - 1)
        if c >= 2:
            drain_gather(c - 2)
    drain_x(NC - 1)
    drain_gather(NC - 2)
    drain_gather(NC - 1)

    for c in range(NC):
        if c % 2 == 0:
            rz0[c].wait_recv()
        else:
            ry[c].wait_recv()
        rfwd[c].wait_recv()
    for ds_ in (rx, ry, rz0, rfwd):
        for d in ds_:
            d.wait_send()

    @functools.partial(pl.run_scoped, exit_sem=pltpu.SemaphoreType.REGULAR)
    def _(exit_sem):
        for p in (x_peer, y_peer, z_peer):
            pl.semaphore_signal(exit_sem, inc=1, device_id=p,
                                device_id_type=pl.DeviceIdType.MESH)
        pl.semaphore_wait(exit_sem, 3)


def kernel(dy, W):
    return pl.pallas_call(
        _body,
        out_shape=jax.ShapeDtypeStruct((M, N), jnp.float32),
        in_specs=[
            pl.BlockSpec(memory_space=pl.ANY),
            pl.BlockSpec(memory_space=pl.ANY),
        ],
        out_specs=pl.BlockSpec(memory_space=pltpu.VMEM),
        scratch_shapes=[
            pltpu.VMEM((BLK, K), jnp.float32),
            pltpu.VMEM((2, CW, K), jnp.float32),
            pltpu.VMEM((BLK, N), jnp.float32),
            pltpu.VMEM((BLK, N), jnp.float32),
            pltpu.SemaphoreType.DMA,
            pltpu.SemaphoreType.DMA((2,)),
            pltpu.SemaphoreType.DMA((NC,)),
            pltpu.SemaphoreType.DMA((NC,)),
            pltpu.SemaphoreType.DMA((NC,)),
            pltpu.SemaphoreType.DMA((NC,)),
            pltpu.SemaphoreType.DMA((NC,)),
            pltpu.SemaphoreType.DMA((NC,)),
            pltpu.SemaphoreType.DMA((NC,)),
            pltpu.SemaphoreType.DMA((NC,)),
        ],
        compiler_params=pltpu.CompilerParams(
            collective_id=0, vmem_limit_bytes=60 * 1024 * 1024),
    )(dy, W)


# device time: 113896 ns/iter; 1.0499x vs baseline; 1.0499x over previous
import functools

import jax
import jax.numpy as jnp
from jax import lax
from jax.experimental import pallas as pl
from jax.experimental.pallas import tpu as pltpu

M = 2048
N = 2048
K = 8192
BLK = M // 4
NC = 8
CW = N // NC


def _body(dy_hbm, w_hbm, out_ref,
          dyv, wbuf, partial_ref, xrecv_ref,
          dysem, wsems,
          xsend_sems, xrecv_sems, ysend_sems, yrecv_sems,
          z0send_sems, z0recv_sems, fsend_sems, frecv_sems):
    xi = lax.axis_index("x")
    yi = lax.axis_index("y")
    zi = lax.axis_index("z")
    r = 2 * yi + zi
    r2 = 2 * (1 - yi) + zi
    r3 = 2 * yi + (1 - zi)
    x_peer = (1 - xi, yi, zi)
    y_peer = (xi, 1 - yi, zi)
    z_peer = (xi, yi, 1 - zi)

    def col(c):
        return pl.ds(c * CW, CW)

    rows_r = pl.ds(r * BLK, BLK)
    rows_r2 = pl.ds(r2 * BLK, BLK)
    rows_r3 = pl.ds(r3 * BLK, BLK)

    dy_cp = pltpu.make_async_copy(dy_hbm.at[rows_r], dyv, dysem)
    dy_cp.start()
    w_cps = [None] * NC

    def start_w(c):
        w_cps[c] = pltpu.make_async_copy(
            w_hbm.at[col(c)], wbuf.at[c % 2], wsems.at[c % 2])
        w_cps[c].start()

    start_w(0)
    start_w(1)

    barrier = pltpu.get_barrier_semaphore()
    for p in (x_peer, y_peer, z_peer):
        pl.semaphore_signal(barrier, inc=1, device_id=p,
                            device_id_type=pl.DeviceIdType.MESH)
    pl.semaphore_wait(barrier, 3)

    dy_cp.wait()

    rx, ry, rz0 = [], [], []
    rfwd = [None] * NC

    def drain_x(c):
        rx[c].wait_recv()
        out_ref[rows_r, col(c)] = (partial_ref[:, col(c)]
                                   + xrecv_ref[:, col(c)])
        d = pltpu.make_async_remote_copy(
            src_ref=out_ref.at[rows_r, col(c)],
            dst_ref=out_ref.at[rows_r, col(c)],
            send_sem=ysend_sems.at[c], recv_sem=yrecv_sems.at[c],
            device_id=y_peer, device_id_type=pl.DeviceIdType.MESH)
        d.start()
        ry.append(d)
        d = pltpu.make_async_remote_copy(
            src_ref=out_ref.at[rows_r, col(c)],
            dst_ref=out_ref.at[rows_r, col(c)],
            send_sem=z0send_sems.at[c], recv_sem=z0recv_sems.at[c],
            device_id=z_peer, device_id_type=pl.DeviceIdType.MESH)
        d.start()
        rz0.append(d)

    def drain_gather(c):
        if c % 2 == 0:
            ry[c].wait_recv()
            d = pltpu.make_async_remote_copy(
                src_ref=out_ref.at[rows_r2, col(c)],
                dst_ref=out_ref.at[rows_r2, col(c)],
                send_sem=fsend_sems.at[c], recv_sem=frecv_sems.at[c],
                device_id=z_peer, device_id_type=pl.DeviceIdType.MESH)
        else:
            rz0[c].wait_recv()
            d = pltpu.make_async_remote_copy(
                src_ref=out_ref.at[rows_r3, col(c)],
                dst_ref=out_ref.at[rows_r3, col(c)],
                send_sem=fsend_sems.at[c], recv_sem=frecv_sems.at[c],
                device_id=y_peer, device_id_type=pl.DeviceIdType.MESH)
        d.start()
        rfwd[c] = d

    for c in range(NC):
        w_cps[c].wait()
        partial_ref[:, col(c)] = lax.dot_general(
            dyv[...], wbuf[c % 2],
            dimension_numbers=(((1,), (1,)), ((), ())),
            preferred_element_type=jnp.float32)
        if c + 2 < NC:
            start_w(c + 2)
        d = pltpu.make_async_remote_copy(
            src_ref=partial_ref.at[:, col(c)],
            dst_ref=xrecv_ref.at[:, col(c)],
            send_sem=xsend_sems.at[c], recv_sem=xrecv_sems.at[c],
            device_id=x_peer, device_id_type=pl.DeviceIdType.MESH)
        d.start()
        rx.append(d)
        if c >= 1:
            drain_x(c - 1)
        if c >= 2:
            drain_gather(c - 2)
    drain_x(NC - 1)
    drain_gather(NC - 2)
    drain_gather(NC - 1)

    for c in range(NC):
        if c % 2 == 0:
            rz0[c].wait_recv()
        else:
            ry[c].wait_recv()
        rfwd[c].wait_recv()
    for ds_ in (rx, ry, rz0, rfwd):
        for d in ds_:
            d.wait_send()

    @functools.partial(pl.run_scoped, exit_sem=pltpu.SemaphoreType.REGULAR)
    def _(exit_sem):
        for p in (x_peer, y_peer, z_peer):
            pl.semaphore_signal(exit_sem, inc=1, device_id=p,
                                device_id_type=pl.DeviceIdType.MESH)
        pl.semaphore_wait(exit_sem, 3)


def kernel(dy, W):
    return pl.pallas_call(
        _body,
        out_shape=jax.ShapeDtypeStruct((M, N), jnp.float32),
        in_specs=[
            pl.BlockSpec(memory_space=pl.ANY),
            pl.BlockSpec(memory_space=pl.ANY),
        ],
        out_specs=pl.BlockSpec(memory_space=pltpu.VMEM),
        scratch_shapes=[
            pltpu.VMEM((BLK, K), jnp.float32),
            pltpu.VMEM((2, CW, K), jnp.float32),
            pltpu.VMEM((BLK, N), jnp.float32),
            pltpu.VMEM((BLK, N), jnp.float32),
            pltpu.SemaphoreType.DMA,
            pltpu.SemaphoreType.DMA((2,)),
            pltpu.SemaphoreType.DMA((NC,)),
            pltpu.SemaphoreType.DMA((NC,)),
            pltpu.SemaphoreType.DMA((NC,)),
            pltpu.SemaphoreType.DMA((NC,)),
            pltpu.SemaphoreType.DMA((NC,)),
            pltpu.SemaphoreType.DMA((NC,)),
            pltpu.SemaphoreType.DMA((NC,)),
            pltpu.SemaphoreType.DMA((NC,)),
        ],
        compiler_params=pltpu.CompilerParams(
            collective_id=0, vmem_limit_bytes=60 * 1024 * 1024),
    )(dy, W)
